# baseline (speedup 1.0000x reference)
import functools

import jax
import jax.numpy as jnp
from jax import lax
from jax.experimental import pallas as pl
from jax.experimental.pallas import tpu as pltpu
from jax.experimental.pallas import tpu_sc as plsc

_BATCH, _SEQ, _EMBED = 4096, 200, 128
_B = _BATCH * _SEQ
_NC, _NS = 2, 16
_NW = _NC * _NS
_BPW = _B // _NW
_CHB = 200
_NCH = _BPW // _CHB
_NB = 4
_NOUT = _NCH // _NB

_mesh = plsc.VectorSubcoreMesh(core_axis_name="c", subcore_axis_name="s")


@functools.partial(
    pl.kernel,
    mesh=_mesh,
    out_type=jax.ShapeDtypeStruct((_B, _EMBED), jnp.float32),
    scratch_types=(
        [pltpu.VMEM((_BPW,), jnp.int32)]
        + [pltpu.VMEM((_CHB, _EMBED), jnp.float32) for _ in range(_NB)]
        + [pltpu.SemaphoreType.DMA for _ in range(2 * _NB)]
    ),
)
def _emb_lookup(idx_hbm, table_hbm, out_hbm, idx_v, *scratch):
    rows = scratch[:_NB]
    gsems = scratch[_NB:2 * _NB]
    ssems = scratch[2 * _NB:]
    wid = lax.axis_index("s") * _NC + lax.axis_index("c")
    base = wid * _BPW
    pltpu.sync_copy(idx_hbm.at[pl.ds(base, _BPW)], idx_v)

    def gather(c, b):
        return pltpu.make_async_copy(
            table_hbm.at[idx_v.at[pl.ds(c * _CHB, _CHB)]], rows[b], gsems[b])

    def store(c, b):
        return pltpu.make_async_copy(
            rows[b], out_hbm.at[pl.ds(base + c * _CHB, _CHB)], ssems[b])

    for b in range(_NB):
        gather(b, b).start()

    def outer(g, carry):
        c0 = g * _NB
        for b in range(_NB):
            gather(c0 + b, b).wait()
            store(c0 + b, b).start()
        for b in range(_NB):
            store(c0 + b, b).wait()
            gather(c0 + _NB + b, b).start()
        return carry

    lax.fori_loop(0, _NOUT - 1, outer, 0)

    cl = (_NOUT - 1) * _NB
    for b in range(_NB):
        gather(cl + b, b).wait()
        store(cl + b, b).start()
    for b in range(_NB):
        store(cl + b, b).wait()


def kernel(data, table):
    idx = data.reshape(_B).astype(jnp.int32)
    out = _emb_lookup(idx, table)
    return out.reshape(_BATCH, _SEQ, _EMBED)


# speedup vs baseline: 1.0148x; 1.0148x over previous
import functools

import jax
import jax.numpy as jnp
from jax import lax
from jax.experimental import pallas as pl
from jax.experimental.pallas import tpu as pltpu
from jax.experimental.pallas import tpu_sc as plsc

_BATCH, _SEQ, _EMBED = 4096, 200, 128
_B = _BATCH * _SEQ
_NC, _NS = 2, 16
_NW = _NC * _NS
_BPW = _B // _NW
_CHB = 200
_NCH = _BPW // _CHB
_NB = 4
_NOUT = _NCH // _NB

_mesh = plsc.VectorSubcoreMesh(core_axis_name="c", subcore_axis_name="s")


@functools.partial(
    pl.kernel,
    mesh=_mesh,
    out_type=jax.ShapeDtypeStruct((_B, _EMBED), jnp.float32),
    scratch_types=(
        [pltpu.VMEM((_BPW,), jnp.int32)]
        + [pltpu.VMEM((_CHB, _EMBED), jnp.float32) for _ in range(_NB)]
        + [pltpu.SemaphoreType.DMA for _ in range(2 * _NB)]
    ),
)
def _emb_lookup(idx_hbm, table_hbm, out_hbm, idx_v, *scratch):
    rows = scratch[:_NB]
    gsems = scratch[_NB:2 * _NB]
    ssems = scratch[2 * _NB:]
    wid = lax.axis_index("s") * _NC + lax.axis_index("c")
    base = wid * _BPW
    pltpu.sync_copy(idx_hbm.at[pl.ds(base, _BPW)], idx_v)

    def gather(c, b):
        return pltpu.make_async_copy(
            table_hbm.at[idx_v.at[pl.ds(c * _CHB, _CHB)]], rows[b], gsems[b])

    def store(c, b):
        return pltpu.make_async_copy(
            rows[b], out_hbm.at[pl.ds(base + c * _CHB, _CHB)], ssems[b])

    for b in range(_NB):
        gather(b, b).start()

    def outer(g, carry):
        for b in range(_NB):
            c = g * _NB + b
            gather(c, b).wait()
            store(c, b).start()
            store(c, b).wait()
            gather(c + _NB, b).start()
        return carry

    lax.fori_loop(0, _NOUT - 1, outer, 0)

    cl = (_NOUT - 1) * _NB
    for b in range(_NB):
        gather(cl + b, b).wait()
        store(cl + b, b).start()
    for b in range(_NB):
        store(cl + b, b).wait()


def kernel(data, table):
    idx = data.reshape(_B).astype(jnp.int32)
    out = _emb_lookup(idx, table)
    return out.reshape(_BATCH, _SEQ, _EMBED)
